# baseline (speedup 1.0000x reference)
import jax
import jax.numpy as jnp
from jax.experimental import pallas as pl

RATIO = 0.1


def _identity_body(x_ref, o_ref):
    o_ref[...] = x_ref[...]


def kernel(feature, labels, feature_mean, feature_var, feature_used):
    feature = pl.pallas_call(
        _identity_body,
        out_shape=jax.ShapeDtypeStruct(feature.shape, feature.dtype),
    )(feature)
    C = feature_mean.shape[0]
    counts = jax.ops.segment_sum(jnp.ones((feature.shape[0],), dtype=jnp.float32), labels, num_segments=C)
    present = counts > 0
    sums = jax.ops.segment_sum(feature, labels, num_segments=C)
    safe_counts = jnp.where(present, counts, 1.0)
    class_mean = sums / safe_counts[:, None]
    used = feature_used != 0
    ema_mean = RATIO * class_mean + (1.0 - RATIO) * feature_mean
    new_mean = jnp.where(present[:, None], jnp.where(used[:, None], ema_mean, class_mean), feature_mean)
    diff = feature - new_mean[labels]
    sqsum = jax.ops.segment_sum(diff * diff, labels, num_segments=C)
    n = counts
    denom = jnp.where(n > 1, n - 1.0, 1.0)
    var_c = jnp.where((n > 1)[:, None], sqsum / denom[:, None], sqsum)
    ema_var = RATIO * var_c + (1.0 - RATIO) * feature_var
    new_var = jnp.where(present[:, None], jnp.where(used[:, None], ema_var, var_c), feature_var)
    new_used = jnp.where(present & jnp.logical_not(used), feature_used + 1.0, feature_used)
    return new_mean, new_var, new_used


# speedup vs baseline: 1.1806x; 1.1806x over previous
import functools

import jax
import jax.numpy as jnp
from jax import lax
from jax.experimental import pallas as pl
from jax.experimental.pallas import tpu as pltpu
from jax.experimental.pallas import tpu_sc as plsc

RATIO = 0.1
CLASS_NUM = 100000
LAST_DIM = 64
BATCH = 16384

NUM_TILES = 32
STEPS = 5
BC = 625
ITEMS_PER_TILE = BATCH // NUM_TILES
GCHUNK = 64
FCHUNK = 128


def _iota16():
    return lax.iota(jnp.int32, 16)


def _splat_i32(x):
    return jnp.full((16,), x, dtype=jnp.int32)


def _sc_body(feat_hbm, lbl_hbm, used_in_hbm, mean_hbm, var_hbm, used_out_hbm,
             LBL, MIDX, SUMF, SQF, COUNT, PCLS, GIDX, FEAT, MSTG, USTG,
             LBLW, UITM, sem0, sem1):
    wid = lax.axis_index("s") * 2 + lax.axis_index("c")
    iota = _iota16()
    lane0 = iota == 0
    zf16 = jnp.zeros((16,), jnp.float32)
    onef16 = jnp.ones((16,), jnp.float32)

    pltpu.sync_copy(lbl_hbm, LBL)

    @pl.loop(0, (BC * LAST_DIM) // 16)
    def _zero_acc(i):
        SUMF[pl.ds(i * 16, 16)] = zf16
        SQF[pl.ds(i * 16, 16)] = zf16

    @pl.loop(0, BATCH // 16)
    def _zero_midx(i):
        MIDX[pl.ds(i * 16, 16)] = jnp.zeros((16,), jnp.int32)

    @pl.loop(0, 640 // 16)
    def _zero_cnt(i):
        COUNT[pl.ds(i * 16, 16)] = zf16
        PCLS[pl.ds(i * 16, 16)] = jnp.zeros((16,), jnp.int32)

    pltpu.sync_copy(lbl_hbm.at[pl.ds(wid * ITEMS_PER_TILE, ITEMS_PER_TILE)],
                    LBLW)
    pltpu.async_copy(used_in_hbm.at[LBLW], UITM, sem0).wait()

    @pl.loop(0, ITEMS_PER_TILE // 16)
    def _bump(i):
        u = UITM[pl.ds(i * 16, 16)]
        UITM[pl.ds(i * 16, 16)] = jnp.where(u != 0.0, u, 1.0)

    pltpu.async_copy(UITM, used_out_hbm.at[LBLW], sem0).wait()

    @pl.loop(0, STEPS)
    def _step(s):
        base = (s * NUM_TILES + wid) * BC
        base_v = jnp.full((16,), base, dtype=jnp.int32)

        def scan_body(v, wp):
            l = LBL[pl.ds(v * 16, 16)]
            m = (l >= base) & (l < base + BC)
            plsc.store_compressed(MIDX.at[pl.ds(wp, 16)], v * 16 + iota,
                                  mask=m)
            cnt = plsc.all_reduce_population_count(m)
            return wp + jnp.max(cnt)

        wp = lax.fori_loop(0, BATCH // 16, scan_body, jnp.int32(0))

        def gchunk_body(g, _):
            off = g * GCHUNK
            pltpu.async_copy(feat_hbm.at[MIDX.at[pl.ds(off, GCHUNK)]],
                             FEAT, sem0).wait()
            nitems = jnp.minimum(GCHUNK, wp - off)

            def item_body(k, _):
                itemv = plsc.load_gather(MIDX, [_splat_i32(off + k)])
                lblv = plsc.load_gather(LBL, [itemv])
                cv = lblv - base_v
                for j in range(4):
                    f = FEAT[k, pl.ds(j * 16, 16)]
                    idx = cv * LAST_DIM + (j * 16 + iota)
                    plsc.addupdate_scatter(SUMF, [idx], f)
                    plsc.addupdate_scatter(SQF, [idx], f * f)
                plsc.addupdate_scatter(COUNT, [cv], onef16, mask=lane0)
                return 0

            lax.fori_loop(0, nitems, item_body, 0)
            return 0

        lax.fori_loop(0, pl.cdiv(wp, GCHUNK), gchunk_body, 0)

        def pres_body(v, np_):
            c = COUNT[pl.ds(v * 16, 16)]
            m = c > 0.0
            plsc.store_compressed(PCLS.at[pl.ds(np_, 16)], v * 16 + iota,
                                  mask=m)
            cnt = plsc.all_reduce_population_count(m)
            return np_ + jnp.max(cnt)

        npres = lax.fori_loop(0, 640 // 16, pres_body, jnp.int32(0))

        def fixup_chunk(off, table_hbm, is_mean):
            chunk = jnp.minimum(FCHUNK, npres - off)
            last_l = plsc.load_gather(PCLS, [_splat_i32(off + chunk - 1)])
            gid_last = last_l + base_v
            for v in range(FCHUNK // 16):
                pv = PCLS[pl.ds(off + v * 16, 16)]
                lane_k = v * 16 + iota
                GIDX[pl.ds(v * 16, 16)] = jnp.where(lane_k < chunk,
                                                    pv + base_v, gid_last)
            pltpu.async_copy(table_hbm.at[GIDX], MSTG, sem0).wait()
            pltpu.async_copy(used_in_hbm.at[GIDX], USTG, sem1).wait()

            def fix_body(k, _):
                cvec = plsc.load_gather(PCLS, [_splat_i32(off + k)])
                n = plsc.load_gather(COUNT, [cvec])
                u = plsc.load_gather(USTG, [_splat_i32(k)])
                for j in range(4):
                    idx = cvec * LAST_DIM + (j * 16 + iota)
                    old = MSTG[k, pl.ds(j * 16, 16)]
                    if is_mean:
                        sv = plsc.load_gather(SUMF, [idx])
                        cm = sv / n
                        nm = jnp.where(u != 0.0,
                                       RATIO * cm + (1.0 - RATIO) * old, cm)
                        MSTG[k, pl.ds(j * 16, 16)] = nm
                        sq = plsc.load_gather(SQF, [idx])
                        sq = sq - 2.0 * nm * sv + n * nm * nm
                        plsc.store_scatter(SQF, [idx], sq)
                        plsc.store_scatter(SUMF, [idx], zf16)
                    else:
                        sq = plsc.load_gather(SQF, [idx])
                        rd = jnp.where(n > 1.0, 1.0 / (n - 1.0), 1.0)
                        vc = sq * rd
                        nv = jnp.where(u != 0.0,
                                       RATIO * vc + (1.0 - RATIO) * old, vc)
                        MSTG[k, pl.ds(j * 16, 16)] = nv
                        plsc.store_scatter(SQF, [idx], zf16)
                if not is_mean:
                    plsc.store_scatter(COUNT, [cvec], zf16, mask=lane0)
                return 0

            lax.fori_loop(0, chunk, fix_body, 0)

            lr = [MSTG[chunk - 1, pl.ds(j * 16, 16)] for j in range(4)]

            def pad_body(p, _):
                for j in range(4):
                    MSTG[p, pl.ds(j * 16, 16)] = lr[j]
                return 0

            lax.fori_loop(chunk, FCHUNK, pad_body, 0)
            pltpu.async_copy(MSTG, table_hbm.at[GIDX], sem0).wait()

        def mean_chunk(ci, _):
            fixup_chunk(ci * FCHUNK, mean_hbm, True)
            return 0

        def var_chunk(ci, _):
            fixup_chunk(ci * FCHUNK, var_hbm, False)
            return 0

        lax.fori_loop(0, pl.cdiv(npres, FCHUNK), mean_chunk, 0)
        lax.fori_loop(0, pl.cdiv(npres, FCHUNK), var_chunk, 0)


@jax.jit
def _run(feature, labels, feature_mean, feature_var, feature_used):
    mean_ref = jax.new_ref(feature_mean)
    var_ref = jax.new_ref(feature_var)
    used_ref = jax.new_ref(feature_used)

    mesh = plsc.VectorSubcoreMesh(core_axis_name="c", subcore_axis_name="s")
    sck = pl.kernel(
        _sc_body,
        out_type=(),
        mesh=mesh,
        compiler_params=pltpu.CompilerParams(
            needs_layout_passes=False, use_tc_tiling_on_sc=False),
        scratch_types=[
            pltpu.VMEM((BATCH,), jnp.int32),
            pltpu.VMEM((BATCH + 16,), jnp.int32),
            pltpu.VMEM((BC * LAST_DIM,), jnp.float32),
            pltpu.VMEM((BC * LAST_DIM,), jnp.float32),
            pltpu.VMEM((640,), jnp.float32),
            pltpu.VMEM((656,), jnp.int32),
            pltpu.VMEM((FCHUNK,), jnp.int32),
            pltpu.VMEM((GCHUNK, LAST_DIM), jnp.float32),
            pltpu.VMEM((FCHUNK, LAST_DIM), jnp.float32),
            pltpu.VMEM((FCHUNK,), jnp.float32),
            pltpu.VMEM((ITEMS_PER_TILE,), jnp.int32),
            pltpu.VMEM((ITEMS_PER_TILE,), jnp.float32),
            pltpu.SemaphoreType.DMA,
            pltpu.SemaphoreType.DMA,
        ],
    )
    sck(feature, labels, feature_used, mean_ref, var_ref, used_ref)
    return mean_ref[...], var_ref[...], used_ref[...]


def kernel(feature, labels, feature_mean, feature_var, feature_used):
    return _run(feature, labels, feature_mean, feature_var, feature_used)


# speedup vs baseline: 1.9507x; 1.6524x over previous
import jax
import jax.numpy as jnp
from jax import lax
from jax.experimental import pallas as pl
from jax.experimental.pallas import tpu as pltpu
from jax.experimental.pallas import tpu_sc as plsc

RATIO = 0.1
CLASS_NUM = 100000
LAST_DIM = 64
BATCH = 16384

NUM_TILES = 32
STEPS = 7
BC = 512
NBLK = 196
LAST_BASE = 99584
VW = 128
MW = 256
GCHUNK = 16
MCAP = 1024
LCH = 2048


def _iota16():
    return lax.iota(jnp.int32, 16)


def _splat_i32(x):
    return jnp.full((16,), x, dtype=jnp.int32)


def _sc_body(feat2_hbm, lbl_hbm, mean_in, var_in, used_in,
             mean_out, var_out, used_out,
             LBLC0, LBLC1, MYL, MIDX, SUMF, SQF, COUNT, PCLS, WOFF,
             MT, VT, FEAT, MSK, USEDB,
             sem_mt, sem_vt, sem_f, sem_u, sem_l0, sem_l1):
    wid = lax.axis_index("s") * 2 + lax.axis_index("c")
    iota = _iota16()
    lane0 = iota == 0
    zf16 = jnp.zeros((16,), jnp.float32)
    onef16 = jnp.ones((16,), jnp.float32)
    wid_v = jnp.full((16,), wid, dtype=jnp.int32)

    lbufs = (LBLC0, LBLC1)
    lsems = (sem_l0, sem_l1)
    pltpu.make_async_copy(lbl_hbm.at[pl.ds(0, LCH)], LBLC0, sem_l0).start()

    @pl.loop(0, (BC * LAST_DIM) // 16)
    def _zero_acc(i):
        SUMF[pl.ds(i * 16, 16)] = zf16
        SQF[pl.ds(i * 16, 16)] = zf16

    @pl.loop(0, BC // 16)
    def _zero_cnt(i):
        COUNT[pl.ds(i * 16, 16)] = zf16

    wpv = jnp.zeros((16,), jnp.int32)
    for ch in range(BATCH // LCH):
        buf = lbufs[ch % 2]
        pltpu.make_async_copy(
            lbl_hbm.at[pl.ds(ch * LCH, LCH)], buf, lsems[ch % 2]).wait()
        if ch + 1 < BATCH // LCH:
            pltpu.make_async_copy(
                lbl_hbm.at[pl.ds((ch + 1) * LCH, LCH)],
                lbufs[(ch + 1) % 2], lsems[(ch + 1) % 2]).start()

        def route_body(v, wpv, _buf=buf, _ch=ch):
            l = _buf[pl.ds(v * 16, 16)]
            owner = lax.shift_right_logical(l, 9) & 31
            m = (owner == wid_v) | ((wid_v == 3) & (l >= LAST_BASE))
            pos = wpv + plsc.cumsum(jnp.where(m, 1, 0)) - 1
            packed = (l * 16384) | (_ch * LCH + v * 16 + iota)
            plsc.store_scatter(MYL, [pos], packed, mask=m)
            return wpv + plsc.all_reduce_population_count(m)

        wpv = lax.fori_loop(0, LCH // 16, route_body, wpv, unroll=4)
    mylen = jnp.max(wpv)

    @pl.loop(0, STEPS)
    def _step(s):
        blk = s * NUM_TILES + wid

        @pl.when(blk < NBLK)
        def _do_step():
            base = jnp.minimum(blk * BC, LAST_BASE)
            base_v = jnp.full((16,), base, dtype=jnp.int32)

            cp_mt = pltpu.make_async_copy(
                mean_in.at[:, pl.ds(base, MW)], MT, sem_mt)
            cp_mt.start()
            cp_u = pltpu.make_async_copy(
                used_in.at[pl.ds(base, BC)], USEDB, sem_u)
            cp_u.start()
            cp_vt0 = pltpu.make_async_copy(
                var_in.at[:, pl.ds(base, VW)], VT, sem_vt)
            cp_vt0.start()

            @pl.loop(0, 16)
            def _fchunk(fc):
                fbase = fc * MCAP

                @pl.when(fbase < mylen)
                def _do_chunk():
                    def filt_body(v, wpv2):
                        gpos = fbase + v * 16
                        p = MYL[pl.ds(gpos, 16)]
                        l = lax.shift_right_logical(p, 14)
                        lc = l - base_v
                        m = ((lc >= 0) & (lc < BC)
                             & ((gpos + iota) < mylen))
                        pos = wpv2 + plsc.cumsum(jnp.where(m, 1, 0)) - 1
                        q = (lc * 16384) | (p & 16383)
                        plsc.store_scatter(MIDX, [pos], q, mask=m)
                        return wpv2 + plsc.all_reduce_population_count(m)

                    wpv2 = lax.fori_loop(0, MCAP // 16, filt_body,
                                         jnp.zeros((16,), jnp.int32),
                                         unroll=4)
                    wp = jnp.max(wpv2)

                    def gchunk_body(g, _):
                        off = g * GCHUNK
                        p16 = MIDX[pl.ds(off, 16)]
                        MSK[pl.ds(0, 16)] = \
                            lax.shift_right_logical(p16 & 16383, 1)
                        pltpu.async_copy(
                            feat2_hbm.at[MSK], FEAT, sem_f).wait()
                        nitems = jnp.minimum(GCHUNK, wp - off)

                        def item_body(k, _):
                            p = plsc.load_gather(MIDX, [_splat_i32(off + k)])
                            cv = lax.shift_right_logical(p, 14)
                            item = p & 16383
                            halfoff = (item & 1) * LAST_DIM
                            ks = _splat_i32(k)
                            for j in range(4):
                                f = plsc.load_gather(
                                    FEAT, [ks, halfoff + (j * 16 + iota)])
                                idx = (j * 16 + iota) * BC + cv
                                plsc.addupdate_scatter(SUMF, [idx], f)
                                plsc.addupdate_scatter(SQF, [idx], f * f)
                            plsc.addupdate_scatter(COUNT, [cv], onef16,
                                                   mask=lane0)
                            return 0

                        lax.fori_loop(0, nitems, item_body, 0)
                        return 0

                    lax.fori_loop(0, pl.cdiv(wp, GCHUNK), gchunk_body, 0)

            WOFF[pl.ds(0, 16)] = jnp.zeros((16,), jnp.int32)

            def pres_body(v, npv):
                c = COUNT[pl.ds(v * 16, 16)]
                m = c > 0.0
                pos = npv + plsc.cumsum(jnp.where(m, 1, 0)) - 1
                plsc.store_scatter(PCLS, [pos], v * 16 + iota, mask=m)
                npv2 = npv + plsc.all_reduce_population_count(m)

                @pl.when((v & 7) == 7)
                def _snap():
                    w1 = lax.shift_right_logical(v, 3) + 1
                    plsc.store_scatter(WOFF, [w1 * 16 + iota], npv2)

                return npv2

            npv = lax.fori_loop(0, BC // 16, pres_body,
                                jnp.zeros((16,), jnp.int32))
            npres = jnp.max(npv)

            cp_u.wait()
            for mw in range(BC // MW):
                if mw > 0:
                    pltpu.make_async_copy(
                        MT, mean_out.at[:, pl.ds(base + (mw - 1) * MW, MW)],
                        sem_mt).wait()
                    cp_mtw = pltpu.make_async_copy(
                        mean_in.at[:, pl.ds(base + mw * MW, MW)], MT, sem_mt)
                    cp_mtw.start()
                    cp_mtw.wait()
                else:
                    cp_mt.wait()

                wr = MW // VW
                mlo = jnp.max(plsc.load_gather(
                    WOFF, [_splat_i32(mw * wr * 16)]))
                mhi = jnp.max(plsc.load_gather(
                    WOFF, [_splat_i32((mw + 1) * wr * 16)]))
                mhi = jnp.minimum(mhi, npres)

                def mean_group(g, _, _mw=mw):
                    k16 = g * 16 + iota
                    cvec = PCLS[pl.ds(g * 16, 16)]
                    cm_ = cvec - _mw * MW
                    valid = (k16 < npres) & (cm_ >= 0) & (cm_ < MW)
                    cidx = jnp.where(valid, cvec, 0)
                    cmw = jnp.where(valid, cm_, 0)
                    n = plsc.load_gather(COUNT, [cidx])
                    u = plsc.load_gather(USEDB, [cidx])
                    used = u != 0.0
                    rn = 1.0 / n

                    def dim_body(d, _):
                        dsp = _splat_i32(d)
                        idx = d * BC + cidx
                        sv = plsc.load_gather(SUMF, [idx])
                        om = plsc.load_gather(MT, [dsp, cmw])
                        cm = sv * rn
                        nm = jnp.where(used,
                                       RATIO * cm + (1.0 - RATIO) * om, cm)
                        plsc.store_scatter(MT, [dsp, cmw], nm, mask=valid)
                        sq = plsc.load_gather(SQF, [idx])
                        sq = sq - 2.0 * nm * sv + n * nm * nm
                        plsc.store_scatter(SQF, [idx], sq, mask=valid)
                        plsc.store_scatter(SUMF, [idx], zf16, mask=valid)
                        return 0

                    lax.fori_loop(0, LAST_DIM, dim_body, 0)
                    return 0

                lax.fori_loop(lax.shift_right_logical(mlo, 4),
                              pl.cdiv(mhi, 16), mean_group, 0)

                pltpu.make_async_copy(
                    MT, mean_out.at[:, pl.ds(base + mw * MW, MW)],
                    sem_mt).start()

            cp_mo = pltpu.make_async_copy(
                MT, mean_out.at[:, pl.ds(base + (BC // MW - 1) * MW, MW)],
                sem_mt)

            for w in range(BC // VW):
                if w > 0:
                    pltpu.make_async_copy(
                        VT, var_out.at[:, pl.ds(base + (w - 1) * VW, VW)],
                        sem_vt).wait()
                    cp_vtw = pltpu.make_async_copy(
                        var_in.at[:, pl.ds(base + w * VW, VW)], VT, sem_vt)
                    cp_vtw.start()
                    cp_vtw.wait()
                else:
                    cp_vt0.wait()

                lo = jnp.max(plsc.load_gather(WOFF, [_splat_i32(w * 16)]))
                hi = jnp.max(plsc.load_gather(WOFF,
                                              [_splat_i32((w + 1) * 16)]))
                hi = jnp.minimum(hi, npres)

                def var_group(g, _, _w=w):
                    k16 = g * 16 + iota
                    cvec = PCLS[pl.ds(g * 16, 16)]
                    cw = cvec - _w * VW
                    valid = (k16 < npres) & (cw >= 0) & (cw < VW)
                    cidx = jnp.where(valid, cvec, 0)
                    cwc = jnp.where(valid, cw, 0)
                    n = plsc.load_gather(COUNT, [cidx])
                    u = plsc.load_gather(USEDB, [cidx])
                    used = u != 0.0
                    rd = jnp.where(n > 1.0, 1.0 / (n - 1.0), 1.0)

                    def dim_body(d, _):
                        dsp = _splat_i32(d)
                        idxs = d * BC + cidx
                        sq = plsc.load_gather(SQF, [idxs])
                        ov = plsc.load_gather(VT, [dsp, cwc])
                        vc = sq * rd
                        nv = jnp.where(used,
                                       RATIO * vc + (1.0 - RATIO) * ov, vc)
                        plsc.store_scatter(VT, [dsp, cwc], nv, mask=valid)
                        plsc.store_scatter(SQF, [idxs], zf16, mask=valid)
                        return 0

                    lax.fori_loop(0, LAST_DIM, dim_body, 0)
                    nu = jnp.where(used, u, 1.0)
                    plsc.store_scatter(USEDB, [cidx], nu, mask=valid)
                    plsc.store_scatter(COUNT, [cidx], zf16, mask=valid)
                    return 0

                lax.fori_loop(lax.shift_right_logical(lo, 4),
                              pl.cdiv(hi, 16), var_group, 0)

                pltpu.make_async_copy(
                    VT, var_out.at[:, pl.ds(base + w * VW, VW)],
                    sem_vt).start()

            pltpu.make_async_copy(
                VT, var_out.at[:, pl.ds(base + (BC // VW - 1) * VW, VW)],
                sem_vt).wait()
            cp_uo = pltpu.make_async_copy(
                USEDB, used_out.at[pl.ds(base, BC)], sem_u)
            cp_uo.start()
            cp_uo.wait()
            cp_mo.wait()


@jax.jit
def _run(feature, labels, feature_mean, feature_var, feature_used):
    mean_t = feature_mean.T
    var_t = feature_var.T
    feat2 = feature.reshape(BATCH // 2, 128)

    mesh = plsc.VectorSubcoreMesh(core_axis_name="c", subcore_axis_name="s")
    sck = pl.kernel(
        _sc_body,
        out_type=(
            jax.ShapeDtypeStruct((LAST_DIM, CLASS_NUM), jnp.float32),
            jax.ShapeDtypeStruct((LAST_DIM, CLASS_NUM), jnp.float32),
            jax.ShapeDtypeStruct((CLASS_NUM,), jnp.float32),
        ),
        mesh=mesh,
        compiler_params=pltpu.CompilerParams(
            needs_layout_passes=False, use_tc_tiling_on_sc=True),
        scratch_types=[
            pltpu.VMEM((LCH,), jnp.int32),
            pltpu.VMEM((LCH,), jnp.int32),
            pltpu.VMEM((BATCH + 16,), jnp.int32),
            pltpu.VMEM((MCAP + 16,), jnp.int32),
            pltpu.VMEM((LAST_DIM * BC,), jnp.float32),
            pltpu.VMEM((LAST_DIM * BC,), jnp.float32),
            pltpu.VMEM((BC,), jnp.float32),
            pltpu.VMEM((BC + 16,), jnp.int32),
            pltpu.VMEM((5 * 16,), jnp.int32),
            pltpu.VMEM((LAST_DIM, MW), jnp.float32),
            pltpu.VMEM((LAST_DIM, VW), jnp.float32),
            pltpu.VMEM((GCHUNK, 128), jnp.float32),
            pltpu.VMEM((16,), jnp.int32),
            pltpu.VMEM((BC,), jnp.float32),
            pltpu.SemaphoreType.DMA,
            pltpu.SemaphoreType.DMA,
            pltpu.SemaphoreType.DMA,
            pltpu.SemaphoreType.DMA,
            pltpu.SemaphoreType.DMA,
            pltpu.SemaphoreType.DMA,
        ],
    )
    nm_t, nv_t, nu = sck(feat2, labels, mean_t, var_t, feature_used)
    return nm_t.T, nv_t.T, nu


def kernel(feature, labels, feature_mean, feature_var, feature_used):
    return _run(feature, labels, feature_mean, feature_var, feature_used)


# speedup vs baseline: 2.0197x; 1.0354x over previous
import jax
import jax.numpy as jnp
from jax import lax
from jax.experimental import pallas as pl
from jax.experimental.pallas import tpu as pltpu
from jax.experimental.pallas import tpu_sc as plsc

RATIO = 0.1
CLASS_NUM = 100000
LAST_DIM = 64
BATCH = 16384

NUM_TILES = 32
STEPS = 7
BC = 512
NBLK = 196
LAST_BASE = 99584
VW = 128
MW = 256
GCHUNK = 32
MCAP = 1024
LCH = 2048


def _iota16():
    return lax.iota(jnp.int32, 16)


def _splat_i32(x):
    return jnp.full((16,), x, dtype=jnp.int32)


def _sc_body(feat2_hbm, lbl_hbm, mean_in, var_in, used_in,
             mean_out, var_out, used_out,
             LBLC0, LBLC1, MYL, MIDX, SUMF, SQF, COUNT, PCLS, WOFF,
             MT, VT, FEAT, MSK, USEDB,
             sem_mt, sem_vt, sem_f, sem_u, sem_l0, sem_l1):
    wid = lax.axis_index("s") * 2 + lax.axis_index("c")
    iota = _iota16()
    lane0 = iota == 0
    zf16 = jnp.zeros((16,), jnp.float32)
    onef16 = jnp.ones((16,), jnp.float32)
    wid_v = jnp.full((16,), wid, dtype=jnp.int32)

    lbufs = (LBLC0, LBLC1)
    lsems = (sem_l0, sem_l1)
    pltpu.make_async_copy(lbl_hbm.at[pl.ds(0, LCH)], LBLC0, sem_l0).start()

    @pl.loop(0, (BC * LAST_DIM) // 16)
    def _zero_acc(i):
        SUMF[pl.ds(i * 16, 16)] = zf16
        SQF[pl.ds(i * 16, 16)] = zf16

    @pl.loop(0, BC // 16)
    def _zero_cnt(i):
        COUNT[pl.ds(i * 16, 16)] = zf16

    wpv = jnp.zeros((16,), jnp.int32)
    for ch in range(BATCH // LCH):
        buf = lbufs[ch % 2]
        pltpu.make_async_copy(
            lbl_hbm.at[pl.ds(ch * LCH, LCH)], buf, lsems[ch % 2]).wait()
        if ch + 1 < BATCH // LCH:
            pltpu.make_async_copy(
                lbl_hbm.at[pl.ds((ch + 1) * LCH, LCH)],
                lbufs[(ch + 1) % 2], lsems[(ch + 1) % 2]).start()

        def route_body(v, wpv, _buf=buf, _ch=ch):
            l = _buf[pl.ds(v * 16, 16)]
            owner = lax.shift_right_logical(l, 9) & 31
            m = (owner == wid_v) | ((wid_v == 3) & (l >= LAST_BASE))
            pos = wpv + plsc.cumsum(jnp.where(m, 1, 0)) - 1
            packed = (l * 16384) | (_ch * LCH + v * 16 + iota)
            plsc.store_scatter(MYL, [pos], packed, mask=m)
            return wpv + plsc.all_reduce_population_count(m)

        wpv = lax.fori_loop(0, LCH // 16, route_body, wpv, unroll=4)
    mylen = jnp.max(wpv)

    @pl.loop(0, STEPS)
    def _step(s):
        blk = s * NUM_TILES + wid

        @pl.when(blk < NBLK)
        def _do_step():
            base = jnp.minimum(blk * BC, LAST_BASE)
            base_v = jnp.full((16,), base, dtype=jnp.int32)

            cp_mt = pltpu.make_async_copy(
                mean_in.at[:, pl.ds(base, MW)], MT, sem_mt)
            cp_mt.start()
            cp_u = pltpu.make_async_copy(
                used_in.at[pl.ds(base, BC)], USEDB, sem_u)
            cp_u.start()
            cp_vt0 = pltpu.make_async_copy(
                var_in.at[:, pl.ds(base, VW)], VT, sem_vt)
            cp_vt0.start()

            @pl.loop(0, 16)
            def _fchunk(fc):
                fbase = fc * MCAP

                @pl.when(fbase < mylen)
                def _do_chunk():
                    def filt_body(v, wpv2):
                        gpos = fbase + v * 16
                        p = MYL[pl.ds(gpos, 16)]
                        l = lax.shift_right_logical(p, 14)
                        lc = l - base_v
                        m = ((lc >= 0) & (lc < BC)
                             & ((gpos + iota) < mylen))
                        pos = wpv2 + plsc.cumsum(jnp.where(m, 1, 0)) - 1
                        q = (lc * 16384) | (p & 16383)
                        plsc.store_scatter(MIDX, [pos], q, mask=m)
                        return wpv2 + plsc.all_reduce_population_count(m)

                    wpv2 = lax.fori_loop(0, MCAP // 16, filt_body,
                                         jnp.zeros((16,), jnp.int32),
                                         unroll=4)
                    wp = jnp.max(wpv2)

                    def gchunk_body(g, _):
                        off = g * GCHUNK
                        for v4 in range(GCHUNK // 16):
                            p16 = MIDX[pl.ds(off + v4 * 16, 16)]
                            MSK[pl.ds(v4 * 16, 16)] = \
                                lax.shift_right_logical(p16 & 16383, 1)
                        pltpu.async_copy(
                            feat2_hbm.at[MSK], FEAT, sem_f).wait()
                        nitems = jnp.minimum(GCHUNK, wp - off)

                        def item_body(k, _):
                            p = plsc.load_gather(MIDX, [_splat_i32(off + k)])
                            cv = lax.shift_right_logical(p, 14)
                            item = p & 16383
                            halfoff = (item & 1) * LAST_DIM
                            ks = _splat_i32(k)
                            for j in range(4):
                                f = plsc.load_gather(
                                    FEAT, [ks, halfoff + (j * 16 + iota)])
                                idx = (j * 16 + iota) * BC + cv
                                plsc.addupdate_scatter(SUMF, [idx], f)
                                plsc.addupdate_scatter(SQF, [idx], f * f)
                            plsc.addupdate_scatter(COUNT, [cv], onef16,
                                                   mask=lane0)
                            return 0

                        lax.fori_loop(0, nitems, item_body, 0)
                        return 0

                    lax.fori_loop(0, pl.cdiv(wp, GCHUNK), gchunk_body, 0)

            WOFF[pl.ds(0, 16)] = jnp.zeros((16,), jnp.int32)

            def pres_body(v, npv):
                c = COUNT[pl.ds(v * 16, 16)]
                m = c > 0.0
                pos = npv + plsc.cumsum(jnp.where(m, 1, 0)) - 1
                plsc.store_scatter(PCLS, [pos], v * 16 + iota, mask=m)
                npv2 = npv + plsc.all_reduce_population_count(m)

                @pl.when((v & 7) == 7)
                def _snap():
                    w1 = lax.shift_right_logical(v, 3) + 1
                    plsc.store_scatter(WOFF, [w1 * 16 + iota], npv2)

                return npv2

            npv = lax.fori_loop(0, BC // 16, pres_body,
                                jnp.zeros((16,), jnp.int32))
            npres = jnp.max(npv)

            cp_u.wait()
            for mw in range(BC // MW):
                if mw > 0:
                    pltpu.make_async_copy(
                        MT, mean_out.at[:, pl.ds(base + (mw - 1) * MW, MW)],
                        sem_mt).wait()
                    cp_mtw = pltpu.make_async_copy(
                        mean_in.at[:, pl.ds(base + mw * MW, MW)], MT, sem_mt)
                    cp_mtw.start()
                    cp_mtw.wait()
                else:
                    cp_mt.wait()

                wr = MW // VW
                mlo = jnp.max(plsc.load_gather(
                    WOFF, [_splat_i32(mw * wr * 16)]))
                mhi = jnp.max(plsc.load_gather(
                    WOFF, [_splat_i32((mw + 1) * wr * 16)]))
                mhi = jnp.minimum(mhi, npres)

                def mean_group(g, _, _mw=mw):
                    k16 = g * 16 + iota
                    cvec = PCLS[pl.ds(g * 16, 16)]
                    cm_ = cvec - _mw * MW
                    valid = (k16 < npres) & (cm_ >= 0) & (cm_ < MW)
                    cidx = jnp.where(valid, cvec, 0)
                    cmw = jnp.where(valid, cm_, 0)
                    n = plsc.load_gather(COUNT, [cidx])
                    u = plsc.load_gather(USEDB, [cidx])
                    used = u != 0.0
                    rn = 1.0 / n

                    def dim_body(d, _):
                        dsp = _splat_i32(d)
                        idx = d * BC + cidx
                        sv = plsc.load_gather(SUMF, [idx])
                        om = plsc.load_gather(MT, [dsp, cmw])
                        cm = sv * rn
                        nm = jnp.where(used,
                                       RATIO * cm + (1.0 - RATIO) * om, cm)
                        plsc.store_scatter(MT, [dsp, cmw], nm, mask=valid)
                        sq = plsc.load_gather(SQF, [idx])
                        sq = sq - 2.0 * nm * sv + n * nm * nm
                        plsc.store_scatter(SQF, [idx], sq, mask=valid)
                        plsc.store_scatter(SUMF, [idx], zf16, mask=valid)
                        return 0

                    lax.fori_loop(0, LAST_DIM, dim_body, 0)
                    return 0

                lax.fori_loop(lax.shift_right_logical(mlo, 4),
                              pl.cdiv(mhi, 16), mean_group, 0)

                pltpu.make_async_copy(
                    MT, mean_out.at[:, pl.ds(base + mw * MW, MW)],
                    sem_mt).start()

            cp_mo = pltpu.make_async_copy(
                MT, mean_out.at[:, pl.ds(base + (BC // MW - 1) * MW, MW)],
                sem_mt)

            for w in range(BC // VW):
                if w > 0:
                    pltpu.make_async_copy(
                        VT, var_out.at[:, pl.ds(base + (w - 1) * VW, VW)],
                        sem_vt).wait()
                    cp_vtw = pltpu.make_async_copy(
                        var_in.at[:, pl.ds(base + w * VW, VW)], VT, sem_vt)
                    cp_vtw.start()
                    cp_vtw.wait()
                else:
                    cp_vt0.wait()

                lo = jnp.max(plsc.load_gather(WOFF, [_splat_i32(w * 16)]))
                hi = jnp.max(plsc.load_gather(WOFF,
                                              [_splat_i32((w + 1) * 16)]))
                hi = jnp.minimum(hi, npres)

                def var_group(g, _, _w=w):
                    k16 = g * 16 + iota
                    cvec = PCLS[pl.ds(g * 16, 16)]
                    cw = cvec - _w * VW
                    valid = (k16 < npres) & (cw >= 0) & (cw < VW)
                    cidx = jnp.where(valid, cvec, 0)
                    cwc = jnp.where(valid, cw, 0)
                    n = plsc.load_gather(COUNT, [cidx])
                    u = plsc.load_gather(USEDB, [cidx])
                    used = u != 0.0
                    rd = jnp.where(n > 1.0, 1.0 / (n - 1.0), 1.0)

                    def dim_body(d, _):
                        dsp = _splat_i32(d)
                        idxs = d * BC + cidx
                        sq = plsc.load_gather(SQF, [idxs])
                        ov = plsc.load_gather(VT, [dsp, cwc])
                        vc = sq * rd
                        nv = jnp.where(used,
                                       RATIO * vc + (1.0 - RATIO) * ov, vc)
                        plsc.store_scatter(VT, [dsp, cwc], nv, mask=valid)
                        plsc.store_scatter(SQF, [idxs], zf16, mask=valid)
                        return 0

                    lax.fori_loop(0, LAST_DIM, dim_body, 0, unroll=4)
                    nu = jnp.where(used, u, 1.0)
                    plsc.store_scatter(USEDB, [cidx], nu, mask=valid)
                    plsc.store_scatter(COUNT, [cidx], zf16, mask=valid)
                    return 0

                lax.fori_loop(lax.shift_right_logical(lo, 4),
                              pl.cdiv(hi, 16), var_group, 0)

                pltpu.make_async_copy(
                    VT, var_out.at[:, pl.ds(base + w * VW, VW)],
                    sem_vt).start()

            pltpu.make_async_copy(
                VT, var_out.at[:, pl.ds(base + (BC // VW - 1) * VW, VW)],
                sem_vt).wait()
            cp_uo = pltpu.make_async_copy(
                USEDB, used_out.at[pl.ds(base, BC)], sem_u)
            cp_uo.start()
            cp_uo.wait()
            cp_mo.wait()


@jax.jit
def _run(feature, labels, feature_mean, feature_var, feature_used):
    mean_t = feature_mean.T
    var_t = feature_var.T
    feat2 = feature.reshape(BATCH // 2, 128)

    mesh = plsc.VectorSubcoreMesh(core_axis_name="c", subcore_axis_name="s")
    sck = pl.kernel(
        _sc_body,
        out_type=(
            jax.ShapeDtypeStruct((LAST_DIM, CLASS_NUM), jnp.float32),
            jax.ShapeDtypeStruct((LAST_DIM, CLASS_NUM), jnp.float32),
            jax.ShapeDtypeStruct((CLASS_NUM,), jnp.float32),
        ),
        mesh=mesh,
        compiler_params=pltpu.CompilerParams(
            needs_layout_passes=False, use_tc_tiling_on_sc=True),
        scratch_types=[
            pltpu.VMEM((LCH,), jnp.int32),
            pltpu.VMEM((LCH,), jnp.int32),
            pltpu.VMEM((BATCH + 16,), jnp.int32),
            pltpu.VMEM((MCAP + 16,), jnp.int32),
            pltpu.VMEM((LAST_DIM * BC,), jnp.float32),
            pltpu.VMEM((LAST_DIM * BC,), jnp.float32),
            pltpu.VMEM((BC,), jnp.float32),
            pltpu.VMEM((BC + 16,), jnp.int32),
            pltpu.VMEM((5 * 16,), jnp.int32),
            pltpu.VMEM((LAST_DIM, MW), jnp.float32),
            pltpu.VMEM((LAST_DIM, VW), jnp.float32),
            pltpu.VMEM((GCHUNK, 128), jnp.float32),
            pltpu.VMEM((GCHUNK,), jnp.int32),
            pltpu.VMEM((BC,), jnp.float32),
            pltpu.SemaphoreType.DMA,
            pltpu.SemaphoreType.DMA,
            pltpu.SemaphoreType.DMA,
            pltpu.SemaphoreType.DMA,
            pltpu.SemaphoreType.DMA,
            pltpu.SemaphoreType.DMA,
        ],
    )
    nm_t, nv_t, nu = sck(feat2, labels, mean_t, var_t, feature_used)
    return nm_t.T, nv_t.T, nu


def kernel(feature, labels, feature_mean, feature_var, feature_used):
    return _run(feature, labels, feature_mean, feature_var, feature_used)


# speedup vs baseline: 2.1761x; 1.0774x over previous
import jax
import jax.numpy as jnp
from jax import lax
from jax.experimental import pallas as pl
from jax.experimental.pallas import tpu as pltpu
from jax.experimental.pallas import tpu_sc as plsc

RATIO = 0.1
CLASS_NUM = 100000
LAST_DIM = 64
BATCH = 16384

NUM_TILES = 32
STEPS = 7
BC = 512
NBLK = 196
LAST_BASE = 99584
VW = 256
SNAP = 128
MW = 256
GCHUNK = 32
MCAP = 1024
LCH = 2048


def _iota16():
    return lax.iota(jnp.int32, 16)


def _splat_i32(x):
    return jnp.full((16,), x, dtype=jnp.int32)


def _sc_body(feat2_hbm, lbl_hbm, mean_in, var_in, used_in,
             mean_out, var_out, used_out,
             LBLC0, LBLC1, MYL, MIDX, SUMF, SQF, COUNT, PCLS, WOFF,
             MT, VT, FEAT, MSK, USEDB,
             sem_mt, sem_vt, sem_f, sem_u, sem_l0, sem_l1):
    wid = lax.axis_index("s") * 2 + lax.axis_index("c")
    iota = _iota16()
    lane0 = iota == 0
    zf16 = jnp.zeros((16,), jnp.float32)
    onef16 = jnp.ones((16,), jnp.float32)
    wid_v = jnp.full((16,), wid, dtype=jnp.int32)

    lbufs = (LBLC0, LBLC1)
    lsems = (sem_l0, sem_l1)
    pltpu.make_async_copy(lbl_hbm.at[pl.ds(0, LCH)], LBLC0, sem_l0).start()

    @pl.loop(0, (BC * LAST_DIM) // 16)
    def _zero_acc(i):
        SUMF[pl.ds(i * 16, 16)] = zf16
        SQF[pl.ds(i * 16, 16)] = zf16

    @pl.loop(0, BC // 16)
    def _zero_cnt(i):
        COUNT[pl.ds(i * 16, 16)] = zf16

    wpv = jnp.zeros((16,), jnp.int32)
    for ch in range(BATCH // LCH):
        buf = lbufs[ch % 2]
        pltpu.make_async_copy(
            lbl_hbm.at[pl.ds(ch * LCH, LCH)], buf, lsems[ch % 2]).wait()
        if ch + 1 < BATCH // LCH:
            pltpu.make_async_copy(
                lbl_hbm.at[pl.ds((ch + 1) * LCH, LCH)],
                lbufs[(ch + 1) % 2], lsems[(ch + 1) % 2]).start()

        def route_body(v, wpv, _buf=buf, _ch=ch):
            l = _buf[pl.ds(v * 16, 16)]
            owner = lax.shift_right_logical(l, 9) & 31
            m = (owner == wid_v) | ((wid_v == 3) & (l >= LAST_BASE))
            pos = wpv + plsc.cumsum(jnp.where(m, 1, 0)) - 1
            packed = (l * 16384) | (_ch * LCH + v * 16 + iota)
            plsc.store_scatter(MYL, [pos], packed, mask=m)
            return wpv + plsc.all_reduce_population_count(m)

        wpv = lax.fori_loop(0, LCH // 16, route_body, wpv, unroll=4)
    mylen = jnp.max(wpv)

    @pl.loop(0, STEPS)
    def _step(s):
        blk = s * NUM_TILES + wid

        @pl.when(blk < NBLK)
        def _do_step():
            base = jnp.minimum(blk * BC, LAST_BASE)
            base_v = jnp.full((16,), base, dtype=jnp.int32)

            cp_mt = pltpu.make_async_copy(
                mean_in.at[:, pl.ds(base, MW)], MT, sem_mt)
            cp_mt.start()
            cp_u = pltpu.make_async_copy(
                used_in.at[pl.ds(base, BC)], USEDB, sem_u)
            cp_u.start()
            cp_vt0 = pltpu.make_async_copy(
                var_in.at[:, pl.ds(base, VW)], VT, sem_vt)
            cp_vt0.start()

            @pl.loop(0, 16)
            def _fchunk(fc):
                fbase = fc * MCAP

                @pl.when(fbase < mylen)
                def _do_chunk():
                    def filt_body(v, wpv2):
                        gpos = fbase + v * 16
                        p = MYL[pl.ds(gpos, 16)]
                        l = lax.shift_right_logical(p, 14)
                        lc = l - base_v
                        m = ((lc >= 0) & (lc < BC)
                             & ((gpos + iota) < mylen))
                        pos = wpv2 + plsc.cumsum(jnp.where(m, 1, 0)) - 1
                        q = (lc * 16384) | (p & 16383)
                        plsc.store_scatter(MIDX, [pos], q, mask=m)
                        return wpv2 + plsc.all_reduce_population_count(m)

                    wpv2 = lax.fori_loop(0, MCAP // 16, filt_body,
                                         jnp.zeros((16,), jnp.int32),
                                         unroll=4)
                    wp = jnp.max(wpv2)

                    def gchunk_body(g, _):
                        off = g * GCHUNK
                        for v4 in range(GCHUNK // 16):
                            p16 = MIDX[pl.ds(off + v4 * 16, 16)]
                            MSK[pl.ds(v4 * 16, 16)] = \
                                lax.shift_right_logical(p16 & 16383, 1)
                        pltpu.async_copy(
                            feat2_hbm.at[MSK], FEAT, sem_f).wait()
                        nitems = jnp.minimum(GCHUNK, wp - off)

                        def item_body(k, _):
                            p = plsc.load_gather(MIDX, [_splat_i32(off + k)])
                            cv = lax.shift_right_logical(p, 14)
                            item = p & 16383
                            halfoff = (item & 1) * LAST_DIM
                            ks = _splat_i32(k)
                            for j in range(4):
                                f = plsc.load_gather(
                                    FEAT, [ks, halfoff + (j * 16 + iota)])
                                idx = (j * 16 + iota) * BC + cv
                                plsc.addupdate_scatter(SUMF, [idx], f)
                                plsc.addupdate_scatter(SQF, [idx], f * f)
                            plsc.addupdate_scatter(COUNT, [cv], onef16,
                                                   mask=lane0)
                            return 0

                        lax.fori_loop(0, nitems, item_body, 0)
                        return 0

                    lax.fori_loop(0, pl.cdiv(wp, GCHUNK), gchunk_body, 0)

            WOFF[pl.ds(0, 16)] = jnp.zeros((16,), jnp.int32)

            def pres_body(v, npv):
                c = COUNT[pl.ds(v * 16, 16)]
                m = c > 0.0
                pos = npv + plsc.cumsum(jnp.where(m, 1, 0)) - 1
                plsc.store_scatter(PCLS, [pos], v * 16 + iota, mask=m)
                npv2 = npv + plsc.all_reduce_population_count(m)

                @pl.when((v & 7) == 7)
                def _snap():
                    w1 = lax.shift_right_logical(v, 3) + 1
                    plsc.store_scatter(WOFF, [w1 * 16 + iota], npv2)

                return npv2

            npv = lax.fori_loop(0, BC // 16, pres_body,
                                jnp.zeros((16,), jnp.int32))
            npres = jnp.max(npv)

            cp_u.wait()
            for mw in range(BC // MW):
                if mw > 0:
                    pltpu.make_async_copy(
                        MT, mean_out.at[:, pl.ds(base + (mw - 1) * MW, MW)],
                        sem_mt).wait()
                    cp_mtw = pltpu.make_async_copy(
                        mean_in.at[:, pl.ds(base + mw * MW, MW)], MT, sem_mt)
                    cp_mtw.start()
                    cp_mtw.wait()
                else:
                    cp_mt.wait()

                wr = MW // SNAP
                mlo = jnp.max(plsc.load_gather(
                    WOFF, [_splat_i32(mw * wr * 16)]))
                mhi = jnp.max(plsc.load_gather(
                    WOFF, [_splat_i32((mw + 1) * wr * 16)]))
                mhi = jnp.minimum(mhi, npres)

                def mean_group(g, _, _mw=mw):
                    k16 = g * 16 + iota
                    cvec = PCLS[pl.ds(g * 16, 16)]
                    cm_ = cvec - _mw * MW
                    valid = (k16 < npres) & (cm_ >= 0) & (cm_ < MW)
                    cidx = jnp.where(valid, cvec, 0)
                    cmw = jnp.where(valid, cm_, 0)
                    n = plsc.load_gather(COUNT, [cidx])
                    u = plsc.load_gather(USEDB, [cidx])
                    used = u != 0.0
                    rn = 1.0 / n

                    def dim_body(d, _):
                        dsp = _splat_i32(d)
                        idx = d * BC + cidx
                        sv = plsc.load_gather(SUMF, [idx])
                        om = plsc.load_gather(MT, [dsp, cmw])
                        cm = sv * rn
                        nm = jnp.where(used,
                                       RATIO * cm + (1.0 - RATIO) * om, cm)
                        plsc.store_scatter(MT, [dsp, cmw], nm, mask=valid)
                        sq = plsc.load_gather(SQF, [idx])
                        sq = sq - 2.0 * nm * sv + n * nm * nm
                        plsc.store_scatter(SQF, [idx], sq, mask=valid)
                        plsc.store_scatter(SUMF, [idx], zf16, mask=valid)
                        return 0

                    lax.fori_loop(0, LAST_DIM, dim_body, 0)
                    return 0

                lax.fori_loop(lax.shift_right_logical(mlo, 4),
                              pl.cdiv(mhi, 16), mean_group, 0)

                pltpu.make_async_copy(
                    MT, mean_out.at[:, pl.ds(base + mw * MW, MW)],
                    sem_mt).start()

            cp_mo = pltpu.make_async_copy(
                MT, mean_out.at[:, pl.ds(base + (BC // MW - 1) * MW, MW)],
                sem_mt)

            for w in range(BC // VW):
                if w > 0:
                    pltpu.make_async_copy(
                        VT, var_out.at[:, pl.ds(base + (w - 1) * VW, VW)],
                        sem_vt).wait()
                    cp_vtw = pltpu.make_async_copy(
                        var_in.at[:, pl.ds(base + w * VW, VW)], VT, sem_vt)
                    cp_vtw.start()
                    cp_vtw.wait()
                else:
                    cp_vt0.wait()

                vwr = VW // SNAP
                lo = jnp.max(plsc.load_gather(
                    WOFF, [_splat_i32(w * vwr * 16)]))
                hi = jnp.max(plsc.load_gather(
                    WOFF, [_splat_i32((w + 1) * vwr * 16)]))
                hi = jnp.minimum(hi, npres)

                def var_group(g, _, _w=w):
                    k16 = g * 16 + iota
                    cvec = PCLS[pl.ds(g * 16, 16)]
                    cw = cvec - _w * VW
                    valid = (k16 < npres) & (cw >= 0) & (cw < VW)
                    cidx = jnp.where(valid, cvec, 0)
                    cwc = jnp.where(valid, cw, 0)
                    n = plsc.load_gather(COUNT, [cidx])
                    u = plsc.load_gather(USEDB, [cidx])
                    used = u != 0.0
                    rd = jnp.where(n > 1.0, 1.0 / (n - 1.0), 1.0)

                    def dim_body(d, _):
                        dsp = _splat_i32(d)
                        idxs = d * BC + cidx
                        sq = plsc.load_gather(SQF, [idxs])
                        ov = plsc.load_gather(VT, [dsp, cwc])
                        vc = sq * rd
                        nv = jnp.where(used,
                                       RATIO * vc + (1.0 - RATIO) * ov, vc)
                        plsc.store_scatter(VT, [dsp, cwc], nv, mask=valid)
                        plsc.store_scatter(SQF, [idxs], zf16, mask=valid)
                        return 0

                    lax.fori_loop(0, LAST_DIM, dim_body, 0, unroll=4)
                    nu = jnp.where(used, u, 1.0)
                    plsc.store_scatter(USEDB, [cidx], nu, mask=valid)
                    plsc.store_scatter(COUNT, [cidx], zf16, mask=valid)
                    return 0

                lax.fori_loop(lax.shift_right_logical(lo, 4),
                              pl.cdiv(hi, 16), var_group, 0)

                pltpu.make_async_copy(
                    VT, var_out.at[:, pl.ds(base + w * VW, VW)],
                    sem_vt).start()

            pltpu.make_async_copy(
                VT, var_out.at[:, pl.ds(base + (BC // VW - 1) * VW, VW)],
                sem_vt).wait()
            cp_uo = pltpu.make_async_copy(
                USEDB, used_out.at[pl.ds(base, BC)], sem_u)
            cp_uo.start()
            cp_uo.wait()
            cp_mo.wait()


@jax.jit
def _run(feature, labels, feature_mean, feature_var, feature_used):
    mean_t = feature_mean.T
    var_t = feature_var.T
    feat2 = feature.reshape(BATCH // 2, 128)

    mesh = plsc.VectorSubcoreMesh(core_axis_name="c", subcore_axis_name="s")
    sck = pl.kernel(
        _sc_body,
        out_type=(
            jax.ShapeDtypeStruct((LAST_DIM, CLASS_NUM), jnp.float32),
            jax.ShapeDtypeStruct((LAST_DIM, CLASS_NUM), jnp.float32),
            jax.ShapeDtypeStruct((CLASS_NUM,), jnp.float32),
        ),
        mesh=mesh,
        compiler_params=pltpu.CompilerParams(
            needs_layout_passes=False, use_tc_tiling_on_sc=True),
        scratch_types=[
            pltpu.VMEM((LCH,), jnp.int32),
            pltpu.VMEM((LCH,), jnp.int32),
            pltpu.VMEM((BATCH + 16,), jnp.int32),
            pltpu.VMEM((MCAP + 16,), jnp.int32),
            pltpu.VMEM((LAST_DIM * BC,), jnp.float32),
            pltpu.VMEM((LAST_DIM * BC,), jnp.float32),
            pltpu.VMEM((BC,), jnp.float32),
            pltpu.VMEM((BC + 16,), jnp.int32),
            pltpu.VMEM((5 * 16,), jnp.int32),
            pltpu.VMEM((LAST_DIM, MW), jnp.float32),
            pltpu.VMEM((LAST_DIM, VW), jnp.float32),
            pltpu.VMEM((GCHUNK, 128), jnp.float32),
            pltpu.VMEM((GCHUNK,), jnp.int32),
            pltpu.VMEM((BC,), jnp.float32),
            pltpu.SemaphoreType.DMA,
            pltpu.SemaphoreType.DMA,
            pltpu.SemaphoreType.DMA,
            pltpu.SemaphoreType.DMA,
            pltpu.SemaphoreType.DMA,
            pltpu.SemaphoreType.DMA,
        ],
    )
    nm_t, nv_t, nu = sck(feat2, labels, mean_t, var_t, feature_used)
    return nm_t.T, nv_t.T, nu


def kernel(feature, labels, feature_mean, feature_var, feature_used):
    return _run(feature, labels, feature_mean, feature_var, feature_used)
